# baseline (speedup 1.0000x reference)
import functools

import jax
import jax.numpy as jnp
import numpy as np
from jax import lax
from jax.experimental import pallas as pl
from jax.experimental.pallas import tpu as pltpu
from jax.experimental.pallas import tpu_sc as plsc

CUTOFF = 5.0
CUTOFF_WIDTH = 0.5
N_BASIS = 16
MP_SCALING = 0.25

E_BLOCK = 8192
NSUB = 16
C_CHUNK = 384

_SEGMENTS = [
    (0, 'W2', 4), (8, 'W2', 5), (16, 'W2', 6), (24, 'W2', 7),
    (32, 'W0', 0), (48, 'W1', 1),
    (64, 'W1', 2), (80, 'W2', 8),
    (96, 'W1', 3),
]


def _build_mats(W0, W1, W2):
    wmap = {'W0': W0 * MP_SCALING, 'W1': W1 * MP_SCALING, 'W2': W2 * MP_SCALING}
    wbig = jnp.zeros((N_BASIS, 128), jnp.float32)
    sel = np.zeros((9, 128), np.float32)
    for col, wname, shi in _SEGMENTS:
        w = wmap[wname]
        wbig = wbig.at[:, col:col + w.shape[1]].set(w)
        sel[shi, col:col + w.shape[1]] = 1.0
    return wbig, jnp.asarray(sel)


def _plane_body(r_ref, sh_ref, w_ref, s_ref, out_ref):
    r = r_ref[:]
    mu = lax.broadcasted_iota(jnp.int32, (1, N_BASIS), 1).astype(
        jnp.float32)[0] * (CUTOFF / (N_BASIS - 1))
    sigma = CUTOFF / N_BASIS
    inner = CUTOFF - CUTOFF_WIDTH
    t = jnp.clip((r - inner) / CUTOFF_WIDTH, 0.0, 1.0)
    cut = 0.5 * (jnp.cos(jnp.pi * t) + 1.0)
    g = jnp.exp(-0.5 * ((r[:, None] - mu[None, :]) / sigma) ** 2) * cut[:, None]
    rbig = jnp.dot(g, w_ref[:], preferred_element_type=jnp.float32,
                   precision=lax.Precision.HIGHEST)
    shbig = jnp.dot(sh_ref[:], s_ref[:], preferred_element_type=jnp.float32,
                    precision=lax.Precision.HIGHEST)
    out_ref[:] = rbig * shbig


@jax.jit
def _planes(r, sh_all, wbig, sel):
    E = r.shape[0]
    grid = (E // E_BLOCK,)
    bs = lambda *dims: pl.BlockSpec((E_BLOCK,) + dims, lambda i: (i,) + (0,) * len(dims))
    return pl.pallas_call(
        _plane_body,
        grid=grid,
        in_specs=[bs(), bs(9),
                  pl.BlockSpec((N_BASIS, 128), lambda i: (0, 0)),
                  pl.BlockSpec((9, 128), lambda i: (0, 0))],
        out_specs=bs(128),
        out_shape=jax.ShapeDtypeStruct((E, 128), jnp.float32),
    )(r, sh_all, wbig, sel)


def _make_sc_kernel(Ep, n_atoms):
    e_per_tile = Ep // NSUB
    n_chunks = e_per_tile // C_CHUNK
    rows_per_tile = n_atoms // NSUB
    nfull = rows_per_tile // C_CHUNK
    rem = rows_per_tile - nfull * C_CHUNK

    mesh = plsc.VectorSubcoreMesh(core_axis_name="c", subcore_axis_name="s")

    @functools.partial(
        pl.kernel, mesh=mesh,
        out_type=jax.ShapeDtypeStruct((2, 2, n_atoms, 32), jnp.bfloat16),
        compiler_params=pltpu.CompilerParams(use_tc_tiling_on_sc=False,
                                             needs_layout_passes=False),
        scratch_types=[
            pltpu.VMEM_SHARED((n_atoms, 32), jnp.bfloat16),
            pltpu.VMEM((2, C_CHUNK), jnp.int32),
            pltpu.VMEM((4, C_CHUNK), jnp.int32),
            pltpu.VMEM((2, C_CHUNK, 16), jnp.float32),
            pltpu.VMEM((2, C_CHUNK, 32), jnp.float32),
            pltpu.VMEM((2, C_CHUNK, 32), jnp.bfloat16),
            pltpu.SemaphoreType.DMA((2,)),
            pltpu.SemaphoreType.DMA((2,)),
            pltpu.SemaphoreType.DMA((2,)),
        ],
    )
    def sc_pass(ce0_hbm, ce1_hbm, neigh_hbm, cent_hbm, stream_hbm,
                out_hbm,
                table, nidx_v, cidx_v, emb_v, a_v, msg_v,
                semL, semG, semS):
        c = lax.axis_index("c")
        s = lax.axis_index("s")
        zero32 = jnp.zeros((32,), jnp.bfloat16)
        row0 = s * rows_per_tile

        def zero_table():
            def _zmsg(e, _):
                msg_v[0, e] = zero32
                return 0
            lax.fori_loop(0, C_CHUNK, _zmsg, 0)
            for j in range(nfull):
                pltpu.sync_copy(msg_v.at[0],
                                table.at[pl.ds(row0 + j * C_CHUNK, C_CHUNK)])
            if rem:
                pltpu.sync_copy(msg_v.at[0, pl.ds(0, rem)],
                                table.at[pl.ds(row0 + nfull * C_CHUNK, rem)])

        def emit_core(ce_hbm, coff):
            def fire_linear(j, p):
                base = s * e_per_tile + j * C_CHUNK
                pltpu.async_copy(neigh_hbm.at[pl.ds(base, C_CHUNK)],
                                 nidx_v.at[p], semL.at[p])
                pltpu.async_copy(cent_hbm.at[pl.ds(base, C_CHUNK)],
                                 cidx_v.at[j % 4], semL.at[p])
                pltpu.async_copy(
                    stream_hbm.at[pl.ds(base, C_CHUNK), pl.ds(coff, 32)],
                    a_v.at[p], semL.at[p])

            def wait_linear(j, p):
                base = s * e_per_tile + j * C_CHUNK
                pltpu.make_async_copy(neigh_hbm.at[pl.ds(base, C_CHUNK)],
                                      nidx_v.at[p], semL.at[p]).wait()
                pltpu.make_async_copy(cent_hbm.at[pl.ds(base, C_CHUNK)],
                                      cidx_v.at[j % 4], semL.at[p]).wait()
                pltpu.make_async_copy(
                    stream_hbm.at[pl.ds(base, C_CHUNK), pl.ds(coff, 32)],
                    a_v.at[p], semL.at[p]).wait()

            def fire_gather(p):
                pltpu.async_copy(ce_hbm.at[nidx_v.at[p]], emb_v.at[p],
                                 semG.at[p])

            def wait_gather(p):
                pltpu.make_async_copy(ce_hbm.at[nidx_v.at[p]], emb_v.at[p],
                                      semG.at[p]).wait()

            def fire_scatter(j, p):
                pltpu.async_copy(msg_v.at[p], table.at[cidx_v.at[j % 4]],
                                 semS.at[p], add=True)

            def drain_scatter(j, p):
                pltpu.make_async_copy(msg_v.at[p], table.at[cidx_v.at[j % 4]],
                                      semS.at[p]).wait()

            fire_linear(0, 0)
            fire_linear(1, 1)
            wait_linear(0, 0)
            fire_gather(0)

            def chunk_body(i, _):
                p = i & 1
                pn = 1 - p

                @pl.when(i >= 2)
                def _():
                    drain_scatter(i - 2, p)
                wait_gather(p)

                def edge_body(e, _):
                    emb = emb_v[p, e]
                    m0 = a_v[p, e, 0:16] * emb
                    m1 = a_v[p, e, 16:32] * emb
                    msg_v[p, e] = plsc.pack(
                        m0, m1, format=plsc.PackFormat.INTERLEAVED)
                    return 0
                lax.fori_loop(0, C_CHUNK, edge_body, 0, unroll=8)
                fire_scatter(i, p)

                @pl.when(i + 1 < n_chunks)
                def _():
                    wait_linear(i + 1, pn)
                    fire_gather(pn)

                @pl.when(i + 2 < n_chunks)
                def _():
                    fire_linear(i + 2, p)
                return 0
            lax.fori_loop(0, n_chunks, chunk_body, 0)

            drain_scatter(n_chunks - 2, (n_chunks - 2) & 1)
            drain_scatter(n_chunks - 1, (n_chunks - 1) & 1)

        rsl = pl.ds(row0, rows_per_tile)

        def run_pass(k, coff_c0, coff_c1, ce_c0, ce_c1):
            zero_table()
            plsc.subcore_barrier()

            @pl.when(c == 0)
            def _():
                emit_core(ce_c0, coff_c0)

            @pl.when(c == 1)
            def _():
                emit_core(ce_c1, coff_c1)

            plsc.subcore_barrier()

            @pl.when(c == 0)
            def _():
                pltpu.sync_copy(table.at[rsl], out_hbm.at[k, 0, rsl])

            @pl.when(c == 1)
            def _():
                pltpu.sync_copy(table.at[rsl], out_hbm.at[k, 1, rsl])

        run_pass(0, 0, 32, ce0_hbm, ce1_hbm)
        plsc.subcore_barrier()
        run_pass(1, 64, 96, ce1_hbm, ce1_hbm)

    return sc_pass


@jax.jit
def _run(r, sh_0, sh_1, sh_2, centers, neighbors, center_embedding, W0, W1, W2):
    n_atoms = center_embedding.shape[0]
    E = r.shape[0]
    step = int(np.lcm(NSUB * C_CHUNK, E_BLOCK))
    Ep = ((E + step - 1) // step) * step
    pad = Ep - E

    wbig, sel = _build_mats(W0, W1, W2)
    rp = jnp.pad(r, (0, pad))
    sh_all = jnp.pad(
        jnp.concatenate([sh_0[:, :, 0], sh_1[:, :, 0], sh_2[:, :, 0]], axis=1),
        ((0, pad), (0, 0)))
    stream = _planes(rp, sh_all, wbig, sel)

    ce = center_embedding[:, 0, :]
    ce_dup = jnp.concatenate([ce[:, :8], ce[:, :8]], axis=1)
    neigh = jnp.pad(neighbors, (0, pad))
    cent = jnp.pad(centers, (0, pad))

    sc_kernel = _make_sc_kernel(Ep, n_atoms)
    pp = sc_kernel(ce_dup, ce, neigh, cent, stream)
    p0, p1 = pp[0], pp[1]

    u0 = p0.astype(jnp.float32)
    u1 = p1.astype(jnp.float32)
    out0 = u0[1, :, 0::2][:, None, :] + center_embedding
    l1m0 = u0[1, :, 1::2][:, :12]
    l1m1 = u1[0, :, 0::2][:, :12]
    l1m2 = u1[1, :, 0::2][:, :12]
    out1 = jnp.stack([l1m0, l1m1, l1m2], axis=1)
    m01 = u0[0, :, 0::2]
    m23 = u0[0, :, 1::2]
    m4 = u1[0, :, 1::2][:, :8]
    out2 = jnp.stack([m01[:, :8], m01[:, 8:], m23[:, :8], m23[:, 8:], m4],
                     axis=1)
    return out0, out1, out2


def kernel(r, sh_0, sh_1, sh_2, centers, neighbors, n_atoms, center_embedding,
           W0, W1, W2):
    return _run(r, sh_0, sh_1, sh_2, centers, neighbors, center_embedding,
                W0, W1, W2)


# speedup vs baseline: 1.4210x; 1.4210x over previous
import functools

import jax
import jax.numpy as jnp
import numpy as np
from jax import lax
from jax.experimental import pallas as pl
from jax.experimental.pallas import tpu as pltpu
from jax.experimental.pallas import tpu_sc as plsc

CUTOFF = 5.0
CUTOFF_WIDTH = 0.5
N_BASIS = 16
MP_SCALING = 0.25

E_BLOCK = 4096
NSUB = 16
C_CHUNK = 512

_SEGMENTS = [
    (0, 'W2', 4), (8, 'W2', 5), (16, 'W2', 6), (24, 'W2', 7),
    (32, 'W0', 0), (48, 'W1', 1),
    (64, 'W1', 2), (80, 'W2', 8),
    (96, 'W1', 3),
]


def _build_mats(W0, W1, W2):
    wmap = {'W0': W0 * MP_SCALING, 'W1': W1 * MP_SCALING, 'W2': W2 * MP_SCALING}
    wbig = jnp.zeros((N_BASIS, 128), jnp.float32)
    colmap = np.zeros((1, 128), np.int32)
    for col, wname, shi in _SEGMENTS:
        w = wmap[wname]
        wbig = wbig.at[:, col:col + w.shape[1]].set(w)
        colmap[0, col:col + w.shape[1]] = shi
    return wbig, jnp.asarray(colmap)


def _plane_body(r_ref, sh_ref, w_ref, s_ref, out_ref):
    r = r_ref[:]
    mu = lax.broadcasted_iota(jnp.int32, (1, N_BASIS), 1).astype(
        jnp.float32)[0] * (CUTOFF / (N_BASIS - 1))
    sigma = CUTOFF / N_BASIS
    inner = CUTOFF - CUTOFF_WIDTH
    t = jnp.clip((r - inner) / CUTOFF_WIDTH, 0.0, 1.0)
    cut = 0.5 * (jnp.cos(jnp.pi * t) + 1.0)
    g = jnp.exp(-0.5 * ((r[:, None] - mu[None, :]) / sigma) ** 2) * cut[:, None]
    rbig = jnp.dot(g, w_ref[:], preferred_element_type=jnp.float32,
                   precision=lax.Precision.HIGHEST)
    idx = jnp.broadcast_to(s_ref[:], (r.shape[0], 128))
    shbig = jnp.take_along_axis(sh_ref[:], idx, axis=1)
    out_ref[:] = rbig * shbig


@jax.jit
def _planes(r, sh_all, wbig, sel):
    E = r.shape[0]
    grid = (E // E_BLOCK,)
    bs = lambda *dims: pl.BlockSpec((E_BLOCK,) + dims, lambda i: (i,) + (0,) * len(dims))
    return pl.pallas_call(
        _plane_body,
        grid=grid,
        in_specs=[bs(), bs(9),
                  pl.BlockSpec((N_BASIS, 128), lambda i: (0, 0)),
                  pl.BlockSpec((1, 128), lambda i: (0, 0))],
        out_specs=bs(128),
        out_shape=jax.ShapeDtypeStruct((E, 128), jnp.float32),
    )(r, sh_all, wbig, sel)


def _make_sc_pass(Ep, n_atoms, coff0, coff1):
    e_per_tile = Ep // NSUB
    n_chunks = e_per_tile // C_CHUNK
    rows_per_tile = n_atoms // NSUB
    nfull = rows_per_tile // C_CHUNK
    rem = rows_per_tile - nfull * C_CHUNK

    mesh = plsc.VectorSubcoreMesh(core_axis_name="c", subcore_axis_name="s")

    @functools.partial(
        pl.kernel, mesh=mesh,
        out_type=jax.ShapeDtypeStruct((2, n_atoms, 32), jnp.bfloat16),
        compiler_params=pltpu.CompilerParams(use_tc_tiling_on_sc=False,
                                             needs_layout_passes=False),
        scratch_types=[
            pltpu.VMEM_SHARED((n_atoms, 32), jnp.bfloat16),
            pltpu.VMEM((2, C_CHUNK), jnp.int32),
            pltpu.VMEM((4, C_CHUNK), jnp.int32),
            pltpu.VMEM((2, C_CHUNK, 16), jnp.float32),
            pltpu.VMEM((2, C_CHUNK, 32), jnp.float32),
            pltpu.VMEM((2, C_CHUNK, 32), jnp.bfloat16),
            pltpu.SemaphoreType.DMA((2,)),
            pltpu.SemaphoreType.DMA((2,)),
            pltpu.SemaphoreType.DMA((2,)),
        ],
    )
    def sc_pass(ce0_hbm, ce1_hbm, neigh_hbm, cent_hbm, stream_hbm,
                out_hbm,
                table, nidx_v, cidx_v, emb_v, a_v, msg_v,
                semL, semG, semS):
        c = lax.axis_index("c")
        s = lax.axis_index("s")
        zero32 = jnp.zeros((32,), jnp.bfloat16)
        row0 = s * rows_per_tile

        def zero_table():
            def _zmsg(e, _):
                msg_v[0, e] = zero32
                return 0
            lax.fori_loop(0, C_CHUNK, _zmsg, 0)
            for j in range(nfull):
                pltpu.sync_copy(msg_v.at[0],
                                table.at[pl.ds(row0 + j * C_CHUNK, C_CHUNK)])
            if rem:
                pltpu.sync_copy(msg_v.at[0, pl.ds(0, rem)],
                                table.at[pl.ds(row0 + nfull * C_CHUNK, rem)])

        def emit_core(ce_hbm, coff):
            def fire_linear(j, p):
                base = s * e_per_tile + j * C_CHUNK
                pltpu.async_copy(neigh_hbm.at[pl.ds(base, C_CHUNK)],
                                 nidx_v.at[p], semL.at[p])
                pltpu.async_copy(cent_hbm.at[pl.ds(base, C_CHUNK)],
                                 cidx_v.at[j % 4], semL.at[p])
                pltpu.async_copy(
                    stream_hbm.at[pl.ds(base, C_CHUNK), pl.ds(coff, 32)],
                    a_v.at[p], semL.at[p])

            def wait_linear(j, p):
                base = s * e_per_tile + j * C_CHUNK
                pltpu.make_async_copy(neigh_hbm.at[pl.ds(base, C_CHUNK)],
                                      nidx_v.at[p], semL.at[p]).wait()
                pltpu.make_async_copy(cent_hbm.at[pl.ds(base, C_CHUNK)],
                                      cidx_v.at[j % 4], semL.at[p]).wait()
                pltpu.make_async_copy(
                    stream_hbm.at[pl.ds(base, C_CHUNK), pl.ds(coff, 32)],
                    a_v.at[p], semL.at[p]).wait()

            def fire_gather(p):
                pltpu.async_copy(ce_hbm.at[nidx_v.at[p]], emb_v.at[p],
                                 semG.at[p])

            def wait_gather(p):
                pltpu.make_async_copy(ce_hbm.at[nidx_v.at[p]], emb_v.at[p],
                                      semG.at[p]).wait()

            def fire_scatter(j, p):
                pltpu.async_copy(msg_v.at[p], table.at[cidx_v.at[j % 4]],
                                 semS.at[p], add=True)

            def drain_scatter(j, p):
                pltpu.make_async_copy(msg_v.at[p], table.at[cidx_v.at[j % 4]],
                                      semS.at[p]).wait()

            fire_linear(0, 0)
            fire_linear(1, 1)
            wait_linear(0, 0)
            fire_gather(0)

            def chunk_body(i, _):
                p = i & 1
                pn = 1 - p

                @pl.when(i >= 2)
                def _():
                    drain_scatter(i - 2, p)
                wait_gather(p)

                def edge_body(e, _):
                    emb = emb_v[p, e]
                    m0 = a_v[p, e, 0:16] * emb
                    m1 = a_v[p, e, 16:32] * emb
                    msg_v[p, e] = plsc.pack(
                        m0, m1, format=plsc.PackFormat.INTERLEAVED)
                    return 0
                lax.fori_loop(0, C_CHUNK, edge_body, 0, unroll=16)
                fire_scatter(i, p)

                @pl.when(i + 1 < n_chunks)
                def _():
                    wait_linear(i + 1, pn)
                    fire_gather(pn)

                @pl.when(i + 2 < n_chunks)
                def _():
                    fire_linear(i + 2, p)
                return 0
            lax.fori_loop(0, n_chunks, chunk_body, 0)

            drain_scatter(n_chunks - 2, (n_chunks - 2) & 1)
            drain_scatter(n_chunks - 1, (n_chunks - 1) & 1)

        rsl = pl.ds(row0, rows_per_tile)

        zero_table()
        plsc.subcore_barrier()

        @pl.when(c == 0)
        def _():
            emit_core(ce0_hbm, coff0)

        @pl.when(c == 1)
        def _():
            emit_core(ce1_hbm, coff1)

        plsc.subcore_barrier()

        @pl.when(c == 0)
        def _():
            pltpu.sync_copy(table.at[rsl], out_hbm.at[0, rsl])

        @pl.when(c == 1)
        def _():
            pltpu.sync_copy(table.at[rsl], out_hbm.at[1, rsl])

    return sc_pass


@jax.jit
def _run(r, sh_0, sh_1, sh_2, centers, neighbors, center_embedding, W0, W1, W2):
    n_atoms = center_embedding.shape[0]
    E = r.shape[0]
    step = int(np.lcm(NSUB * C_CHUNK, E_BLOCK))
    Ep = ((E + step - 1) // step) * step
    pad = Ep - E

    wbig, sel = _build_mats(W0, W1, W2)
    rp = jnp.pad(r, (0, pad))
    sh_all = jnp.pad(
        jnp.concatenate([sh_0[:, :, 0], sh_1[:, :, 0], sh_2[:, :, 0]], axis=1),
        ((0, pad), (0, 0)))
    stream = _planes(rp, sh_all, wbig, sel)

    ce = center_embedding[:, 0, :]
    ce_dup = jnp.concatenate([ce[:, :8], ce[:, :8]], axis=1)
    neigh = jnp.pad(neighbors, (0, pad))
    cent = jnp.pad(centers, (0, pad))

    pass0 = _make_sc_pass(Ep, n_atoms, 0, 32)
    pass1 = _make_sc_pass(Ep, n_atoms, 64, 96)
    p0 = pass0(ce_dup, ce, neigh, cent, stream)
    p1 = pass1(ce, ce, neigh, cent, stream)

    u0 = p0.astype(jnp.float32)
    u1 = p1.astype(jnp.float32)
    out0 = u0[1, :, 0::2][:, None, :] + center_embedding
    l1m0 = u0[1, :, 1::2][:, :12]
    l1m1 = u1[0, :, 0::2][:, :12]
    l1m2 = u1[1, :, 0::2][:, :12]
    out1 = jnp.stack([l1m0, l1m1, l1m2], axis=1)
    m01 = u0[0, :, 0::2]
    m23 = u0[0, :, 1::2]
    m4 = u1[0, :, 1::2][:, :8]
    out2 = jnp.stack([m01[:, :8], m01[:, 8:], m23[:, :8], m23[:, 8:], m4],
                     axis=1)
    return out0, out1, out2


def kernel(r, sh_0, sh_1, sh_2, centers, neighbors, n_atoms, center_embedding,
           W0, W1, W2):
    return _run(r, sh_0, sh_1, sh_2, centers, neighbors, center_embedding,
                W0, W1, W2)
